# baseline (speedup 1.0000x reference)
import jax
import jax.numpy as jnp
from jax import lax
from jax.experimental import pallas as pl
from jax.experimental.pallas import tpu as pltpu
from jax.experimental.pallas import tpu_sc as plsc

HID = 1024
NE = 8
INTER = 2752
T = 4096
TOPK = 2

BM = 256
NB = 40
PADT = NB * BM
BI = 512
IB = (INTER + BI - 1) // BI
LAST_VALID = INTER - (IB - 1) * BI

SC_NC = 2
SC_NS = 16
SC_NW = SC_NC * SC_NS
BPW = PADT // SC_NW
CH = 160
NCHUNK = BPW // CH
HIDW = HID // 2


def _routing_body(x_ref, wg_ref, sel_ref, wts_ref):
    logits = jnp.dot(x_ref[...], wg_ref[...],
                     preferred_element_type=jnp.float32)
    eids = jax.lax.broadcasted_iota(jnp.int32, logits.shape, 1)
    m1 = jnp.max(logits, axis=1, keepdims=True)
    e1 = jnp.min(jnp.where(logits == m1, eids, NE), axis=1, keepdims=True)
    l2m = jnp.where(eids == e1, -jnp.inf, logits)
    m2 = jnp.max(l2m, axis=1, keepdims=True)
    e2 = jnp.min(jnp.where(l2m == m2, eids, NE), axis=1, keepdims=True)
    wa = jax.lax.logistic(m1 - m2)
    sel_ref[...] = jnp.concatenate([e1, e2], axis=1)
    wts_ref[...] = jnp.concatenate([wa, 1.0 - wa], axis=1)


def _gather_body(xb_hbm, src_hbm, out_hbm, idx_v, rows_v, sem):
    wid = lax.axis_index("s") * SC_NC + lax.axis_index("c")
    base = wid * BPW
    for j in range(NCHUNK):
        pltpu.sync_copy(src_hbm.at[pl.ds(base + j * CH, CH)], idx_v)
        pltpu.async_copy(xb_hbm.at[idx_v], rows_v, sem).wait()
        pltpu.sync_copy(rows_v, out_hbm.at[pl.ds(base + j * CH, CH)])


_gather_rows = pl.kernel(
    _gather_body,
    out_type=jax.ShapeDtypeStruct((PADT, HIDW), jnp.int32),
    mesh=plsc.VectorSubcoreMesh(core_axis_name="c", subcore_axis_name="s"),
    scratch_types=[
        pltpu.VMEM((CH,), jnp.int32),
        pltpu.VMEM((CH, HIDW), jnp.int32),
        pltpu.SemaphoreType.DMA,
    ],
)


def _ffn_body(be_ref, xg_ref, w1_ref, w3_ref, w2_ref, ws_ref, h_ref,
              w1s_ref, w3s_ref, w2s_ref):
    ib = pl.program_id(0)
    nb = pl.program_id(1)
    prev = be_ref[jnp.maximum(nb - 1, 0)]
    is_new = jnp.logical_or(nb == 0, be_ref[nb] != prev)
    valid = jnp.where(ib == IB - 1, LAST_VALID, BI)

    @pl.when(is_new)
    def _():
        w1s_ref[...] = w1_ref[0].astype(jnp.bfloat16)
        w3s_ref[...] = w3_ref[0].astype(jnp.bfloat16)
        w2 = w2_ref[0]
        wrow = jax.lax.broadcasted_iota(jnp.int32, w2.shape, 0)
        w2s_ref[...] = jnp.where(wrow < valid, w2, 0.0).astype(jnp.bfloat16)

    x = xg_ref[...]
    a = jnp.dot(x, w1s_ref[...], preferred_element_type=jnp.float32)
    b = jnp.dot(x, w3s_ref[...], preferred_element_type=jnp.float32)
    g = a * jax.lax.logistic(a) * b
    gcol = jax.lax.broadcasted_iota(jnp.int32, g.shape, 1)
    g = jnp.where(gcol < valid, g, 0.0).astype(jnp.bfloat16)
    h = jnp.dot(g, w2s_ref[...], preferred_element_type=jnp.float32)
    rows = pl.ds(nb * BM, BM)

    @pl.when(ib == 0)
    def _():
        h_ref[rows, :] = h

    @pl.when(ib > 0)
    def _():
        h_ref[rows, :] += h

    @pl.when(ib == IB - 1)
    def _():
        h_ref[rows, :] *= ws_ref[0]


def kernel(hidden_states, Wg, W1, W2, W3):
    bs, seq, hid = hidden_states.shape
    x = hidden_states.reshape(-1, hid)

    sel, wts = pl.pallas_call(
        _routing_body,
        grid=(1,),
        in_specs=[
            pl.BlockSpec((T, HID), lambda i: (0, 0)),
            pl.BlockSpec((HID, NE), lambda i: (0, 0)),
        ],
        out_specs=[
            pl.BlockSpec((T, TOPK), lambda i: (0, 0)),
            pl.BlockSpec((T, TOPK), lambda i: (0, 0)),
        ],
        out_shape=[
            jax.ShapeDtypeStruct((T, TOPK), jnp.int32),
            jax.ShapeDtypeStruct((T, TOPK), jnp.float32),
        ],
    )(x, Wg)

    fe = sel.reshape(-1)
    tokf = (jnp.arange(T * TOPK, dtype=jnp.int32) // TOPK)
    wflat = wts.reshape(-1)
    oh = (fe[:, None] == jnp.arange(NE)[None, :]).astype(jnp.int32)
    csum = jnp.cumsum(oh, axis=0)
    rank = jnp.sum((csum - oh) * oh, axis=1)
    counts = csum[-1]
    plain_start = jnp.concatenate(
        [jnp.zeros((1,), jnp.int32), jnp.cumsum(counts)])[:NE]
    seg = -(-counts // BM) * BM
    astart = jnp.concatenate(
        [jnp.zeros((1,), jnp.int32), jnp.cumsum(seg)])[:NE]
    pos = astart[fe] + rank

    _, sorted_tok, sorted_w = jax.lax.sort(
        (fe, tokf, wflat), num_keys=1, is_stable=True)
    r = jnp.arange(PADT, dtype=jnp.int32)
    er = (jnp.sum(astart[None, :] <= r[:, None], axis=1)
          .astype(jnp.int32) - 1)
    cr = jnp.clip(plain_start[er] + r - astart[er], 0, T * TOPK - 1)
    src = sorted_tok[cr]
    wrow = sorted_w[cr]
    block_rows = jnp.arange(NB, dtype=jnp.int32) * BM
    be = (jnp.sum(astart[None, :] <= block_rows[:, None], axis=1)
          .astype(jnp.int32) - 1)

    xb32 = lax.bitcast_convert_type(
        x.astype(jnp.bfloat16).reshape(T, HIDW, 2), jnp.int32)
    xg32 = _gather_rows(xb32, src)
    xg = lax.bitcast_convert_type(xg32, jnp.bfloat16).reshape(PADT, HID)
    ws3 = wrow.reshape(NB, BM, 1)

    h = pl.pallas_call(
        _ffn_body,
        grid_spec=pltpu.PrefetchScalarGridSpec(
            num_scalar_prefetch=1,
            grid=(IB, NB),
            in_specs=[
                pl.BlockSpec((BM, HID), lambda ib, nb, be_s: (nb, 0)),
                pl.BlockSpec((1, HID, BI), lambda ib, nb, be_s: (be_s[nb], 0, ib)),
                pl.BlockSpec((1, HID, BI), lambda ib, nb, be_s: (be_s[nb], 0, ib)),
                pl.BlockSpec((1, BI, HID), lambda ib, nb, be_s: (be_s[nb], ib, 0)),
                pl.BlockSpec((1, BM, 1), lambda ib, nb, be_s: (nb, 0, 0)),
            ],
            out_specs=pl.BlockSpec((PADT, HID), lambda ib, nb, be_s: (0, 0)),
            scratch_shapes=[
                pltpu.VMEM((HID, BI), jnp.bfloat16),
                pltpu.VMEM((HID, BI), jnp.bfloat16),
                pltpu.VMEM((BI, HID), jnp.bfloat16),
            ],
        ),
        out_shape=jax.ShapeDtypeStruct((PADT, HID), jnp.float32),
        compiler_params=pltpu.CompilerParams(
            dimension_semantics=("arbitrary", "arbitrary"),
        ),
    )(be, xg, W1, W3, W2, ws3)

    p = pos.reshape(T, TOPK)
    out = h[p[:, 0]] + h[p[:, 1]]
    return out.reshape(bs, seq, hid)


# speedup vs baseline: 1.3976x; 1.3976x over previous
import jax
import jax.numpy as jnp
from jax import lax
from jax.experimental import pallas as pl
from jax.experimental.pallas import tpu as pltpu
from jax.experimental.pallas import tpu_sc as plsc

HID = 1024
NE = 8
INTER = 2752
T = 4096
TOPK = 2

BM = 256
NB = 40
PADT = NB * BM
BI = 512
IB = (INTER + BI - 1) // BI
LAST_VALID = INTER - (IB - 1) * BI

SC_NC = 2
SC_NS = 16
SC_NW = SC_NC * SC_NS
BPW = PADT // SC_NW
CH = 80
NCHUNK = BPW // CH


def _routing_body(x_ref, wg_ref, sel_ref, wts_ref):
    logits = jnp.dot(x_ref[...], wg_ref[...],
                     preferred_element_type=jnp.float32)
    eids = jax.lax.broadcasted_iota(jnp.int32, logits.shape, 1)
    m1 = jnp.max(logits, axis=1, keepdims=True)
    e1 = jnp.min(jnp.where(logits == m1, eids, NE), axis=1, keepdims=True)
    l2m = jnp.where(eids == e1, -jnp.inf, logits)
    m2 = jnp.max(l2m, axis=1, keepdims=True)
    e2 = jnp.min(jnp.where(l2m == m2, eids, NE), axis=1, keepdims=True)
    wa = jax.lax.logistic(m1 - m2)
    sel_ref[...] = jnp.concatenate([e1, e2], axis=1)
    wts_ref[...] = jnp.concatenate([wa, 1.0 - wa], axis=1)


def _gather_body(x_hbm, st_hbm, cr_hbm, out_hbm, idx_v, src_v, rows_v, sem):
    wid = lax.axis_index("s") * SC_NC + lax.axis_index("c")
    base = wid * BPW
    for j in range(NCHUNK):
        pltpu.sync_copy(cr_hbm.at[pl.ds(base + j * CH, CH)], idx_v)
        pltpu.async_copy(st_hbm.at[idx_v], src_v, sem).wait()
        pltpu.async_copy(x_hbm.at[src_v], rows_v, sem).wait()
        pltpu.sync_copy(rows_v, out_hbm.at[pl.ds(base + j * CH, CH)])


_gather_rows = pl.kernel(
    _gather_body,
    out_type=jax.ShapeDtypeStruct((PADT, HID), jnp.float32),
    mesh=plsc.VectorSubcoreMesh(core_axis_name="c", subcore_axis_name="s"),
    scratch_types=[
        pltpu.VMEM((CH,), jnp.int32),
        pltpu.VMEM((CH,), jnp.int32),
        pltpu.VMEM((CH, HID), jnp.float32),
        pltpu.SemaphoreType.DMA,
    ],
)


def _ffn_body(be_ref, xg_ref, w1_ref, w3_ref, w2_ref, h_ref,
              w1s_ref, w3s_ref, w2s_ref):
    ib = pl.program_id(0)
    nb = pl.program_id(1)
    prev = be_ref[jnp.maximum(nb - 1, 0)]
    is_new = jnp.logical_or(nb == 0, be_ref[nb] != prev)
    valid = jnp.where(ib == IB - 1, LAST_VALID, BI)

    @pl.when(is_new)
    def _():
        w1s_ref[...] = w1_ref[0].astype(jnp.bfloat16)
        w3s_ref[...] = w3_ref[0].astype(jnp.bfloat16)
        w2 = w2_ref[0]
        wrow = jax.lax.broadcasted_iota(jnp.int32, w2.shape, 0)
        w2s_ref[...] = jnp.where(wrow < valid, w2, 0.0).astype(jnp.bfloat16)

    x = xg_ref[...].astype(jnp.bfloat16)
    a = jnp.dot(x, w1s_ref[...], preferred_element_type=jnp.float32)
    b = jnp.dot(x, w3s_ref[...], preferred_element_type=jnp.float32)
    g = a * jax.lax.logistic(a) * b
    gcol = jax.lax.broadcasted_iota(jnp.int32, g.shape, 1)
    g = jnp.where(gcol < valid, g, 0.0).astype(jnp.bfloat16)
    h = jnp.dot(g, w2s_ref[...], preferred_element_type=jnp.float32)
    rows = pl.ds(nb * BM, BM)

    @pl.when(ib == 0)
    def _():
        h_ref[rows, :] = h

    @pl.when(ib > 0)
    def _():
        h_ref[rows, :] += h


def kernel(hidden_states, Wg, W1, W2, W3):
    bs, seq, hid = hidden_states.shape
    x = hidden_states.reshape(-1, hid)

    sel, wts = pl.pallas_call(
        _routing_body,
        grid=(1,),
        in_specs=[
            pl.BlockSpec((T, HID), lambda i: (0, 0)),
            pl.BlockSpec((HID, NE), lambda i: (0, 0)),
        ],
        out_specs=[
            pl.BlockSpec((T, TOPK), lambda i: (0, 0)),
            pl.BlockSpec((T, TOPK), lambda i: (0, 0)),
        ],
        out_shape=[
            jax.ShapeDtypeStruct((T, TOPK), jnp.int32),
            jax.ShapeDtypeStruct((T, TOPK), jnp.float32),
        ],
    )(x, Wg)

    fe = sel.reshape(-1)
    tokf = (jnp.arange(T * TOPK, dtype=jnp.int32) // TOPK)
    oh = (fe[:, None] == jnp.arange(NE)[None, :]).astype(jnp.int32)
    csum = jnp.cumsum(oh, axis=0)
    rank = jnp.sum((csum - oh) * oh, axis=1)
    counts = csum[-1]
    plain_start = jnp.concatenate(
        [jnp.zeros((1,), jnp.int32), jnp.cumsum(counts)])[:NE]
    seg = -(-counts // BM) * BM
    astart = jnp.concatenate(
        [jnp.zeros((1,), jnp.int32), jnp.cumsum(seg)])[:NE]
    pos = astart[fe] + rank

    _, sorted_tok = jax.lax.sort((fe, tokf), num_keys=1, is_stable=True)
    r = jnp.arange(PADT, dtype=jnp.int32)
    er = (jnp.sum(astart[None, :] <= r[:, None], axis=1)
          .astype(jnp.int32) - 1)
    cr = jnp.clip(plain_start[er] + r - astart[er], 0, T * TOPK - 1)
    block_rows = jnp.arange(NB, dtype=jnp.int32) * BM
    be = (jnp.sum(astart[None, :] <= block_rows[:, None], axis=1)
          .astype(jnp.int32) - 1)

    xg = _gather_rows(x, sorted_tok, cr)

    h = pl.pallas_call(
        _ffn_body,
        grid_spec=pltpu.PrefetchScalarGridSpec(
            num_scalar_prefetch=1,
            grid=(IB, NB),
            in_specs=[
                pl.BlockSpec((BM, HID), lambda ib, nb, be_s: (nb, 0)),
                pl.BlockSpec((1, HID, BI), lambda ib, nb, be_s: (be_s[nb], 0, ib)),
                pl.BlockSpec((1, HID, BI), lambda ib, nb, be_s: (be_s[nb], 0, ib)),
                pl.BlockSpec((1, BI, HID), lambda ib, nb, be_s: (be_s[nb], ib, 0)),
            ],
            out_specs=pl.BlockSpec((PADT, HID), lambda ib, nb, be_s: (0, 0)),
            scratch_shapes=[
                pltpu.VMEM((HID, BI), jnp.bfloat16),
                pltpu.VMEM((HID, BI), jnp.bfloat16),
                pltpu.VMEM((BI, HID), jnp.bfloat16),
            ],
        ),
        out_shape=jax.ShapeDtypeStruct((PADT, HID), jnp.float32),
        compiler_params=pltpu.CompilerParams(
            dimension_semantics=("arbitrary", "arbitrary"),
        ),
    )(be, xg, W1, W3, W2)

    p = pos.reshape(T, TOPK)
    out = wts[:, :1] * h[p[:, 0]] + wts[:, 1:] * h[p[:, 1]]
    return out.reshape(bs, seq, hid)
